# baseline (device time: 11178 ns/iter reference)
import jax
import jax.numpy as jnp
from jax import lax
from jax.experimental import pallas as pl
from jax.experimental.pallas import tpu as pltpu

FWD = 160
FP = 5
FS = FWD // FP
KEEP = 192
KP = 2
KS = KEEP // KP
KBASE = 160
SENT = FWD + KEEP


def kernel(x):
    m, n = x.shape
    assert FWD + KEEP + FWD == m

    def body(x_ref, out_ref, xb, other_buf,
             xf_send, xf_recv, xk_send, xk_recv, yf_send, yf_recv):
        my_x = lax.axis_index("x")
        my_y = lax.axis_index("y")
        other_x = 1 - my_x
        other_y = 1 - my_y
        fb = my_y * (KBASE + KEEP)
        ifb = other_y * (KBASE + KEEP)
        sbase = my_y * FWD
        obase = other_x * m

        barrier_sem = pltpu.get_barrier_semaphore()
        pl.semaphore_signal(
            barrier_sem, inc=1,
            device_id=(other_x, my_y), device_id_type=pl.DeviceIdType.MESH,
        )
        pl.semaphore_signal(
            barrier_sem, inc=1,
            device_id=(my_x, other_y), device_id_type=pl.DeviceIdType.MESH,
        )

        xb[...] = x_ref[pl.ds(sbase, SENT), :].astype(jnp.bfloat16)

        pl.semaphore_wait(barrier_sem, 2)

        xf_rdmas = []
        for p in range(FP):
            rdma = pltpu.make_async_remote_copy(
                src_ref=xb.at[pl.ds(fb - sbase + p * FS, FS)],
                dst_ref=other_buf.at[pl.ds(fb + p * FS, FS)],
                send_sem=xf_send.at[p],
                recv_sem=xf_recv.at[p],
                device_id=(other_x, my_y),
                device_id_type=pl.DeviceIdType.MESH,
            )
            rdma.start()
            xf_rdmas.append(rdma)
        xk_rdmas = []
        for p in range(KP):
            rdma = pltpu.make_async_remote_copy(
                src_ref=xb.at[pl.ds(KBASE - sbase + p * KS, KS)],
                dst_ref=other_buf.at[pl.ds(KBASE + p * KS, KS)],
                send_sem=xk_send.at[p],
                recv_sem=xk_recv.at[p],
                device_id=(other_x, my_y),
                device_id_type=pl.DeviceIdType.MESH,
            )
            rdma.start()
            xk_rdmas.append(rdma)

        out_ref[pl.ds(my_x * m, m), :] = x_ref[...].astype(jnp.bfloat16)

        yf_rdmas = []
        for p in range(FP):
            xf_rdmas[p].wait()
            rdma = pltpu.make_async_remote_copy(
                src_ref=other_buf.at[pl.ds(fb + p * FS, FS)],
                dst_ref=other_buf.at[pl.ds(fb + p * FS, FS)],
                send_sem=yf_send.at[p],
                recv_sem=yf_recv.at[p],
                device_id=(my_x, other_y),
                device_id_type=pl.DeviceIdType.MESH,
            )
            rdma.start()
            yf_rdmas.append(rdma)
            out_ref[pl.ds(obase + fb + p * FS, FS), :] = \
                other_buf[pl.ds(fb + p * FS, FS), :]

        for p in range(KP):
            xk_rdmas[p].wait()
            out_ref[pl.ds(obase + KBASE + p * KS, KS), :] = \
                other_buf[pl.ds(KBASE + p * KS, KS), :]

        for p in range(FP):
            yf_rdmas[p].wait()
            out_ref[pl.ds(obase + ifb + p * FS, FS), :] = \
                other_buf[pl.ds(ifb + p * FS, FS), :]

    return pl.pallas_call(
        body,
        out_shape=jax.ShapeDtypeStruct((2 * m, n), jnp.bfloat16),
        in_specs=[pl.BlockSpec(memory_space=pltpu.VMEM)],
        out_specs=pl.BlockSpec(memory_space=pltpu.VMEM),
        scratch_shapes=[
            pltpu.VMEM((SENT, n), jnp.bfloat16),
            pltpu.VMEM((m, n), jnp.bfloat16),
            pltpu.SemaphoreType.DMA((FP,)),
            pltpu.SemaphoreType.DMA((FP,)),
            pltpu.SemaphoreType.DMA((KP,)),
            pltpu.SemaphoreType.DMA((KP,)),
            pltpu.SemaphoreType.DMA((FP,)),
            pltpu.SemaphoreType.DMA((FP,)),
        ],
        compiler_params=pltpu.CompilerParams(collective_id=0),
    )(x)


# device time: 11110 ns/iter; 1.0061x vs baseline; 1.0061x over previous
import jax
import jax.numpy as jnp
from jax import lax
from jax.experimental import pallas as pl
from jax.experimental.pallas import tpu as pltpu

FWD = 160
FP = 5
FS = FWD // FP
KEEP = 192
KP = 2
KS = KEEP // KP
KBASE = 160
SENT = FWD + KEEP


def kernel(x):
    m, n = x.shape
    assert FWD + KEEP + FWD == m

    def body(x_ref, out_ref, xb,
             xf_send, xf_recv, xk_send, xk_recv, yf_send, yf_recv):
        my_x = lax.axis_index("x")
        my_y = lax.axis_index("y")
        other_x = 1 - my_x
        other_y = 1 - my_y
        fb = my_y * (KBASE + KEEP)
        sbase = my_y * FWD
        obase = other_x * m

        barrier_sem = pltpu.get_barrier_semaphore()
        pl.semaphore_signal(
            barrier_sem, inc=1,
            device_id=(other_x, my_y), device_id_type=pl.DeviceIdType.MESH,
        )
        pl.semaphore_signal(
            barrier_sem, inc=1,
            device_id=(my_x, other_y), device_id_type=pl.DeviceIdType.MESH,
        )

        xb[...] = x_ref[pl.ds(sbase, SENT), :].astype(jnp.bfloat16)

        pl.semaphore_wait(barrier_sem, 2)

        xf_rdmas = []
        for p in range(FP):
            rdma = pltpu.make_async_remote_copy(
                src_ref=xb.at[pl.ds(fb - sbase + p * FS, FS)],
                dst_ref=out_ref.at[pl.ds(my_x * m + fb + p * FS, FS)],
                send_sem=xf_send.at[p],
                recv_sem=xf_recv.at[p],
                device_id=(other_x, my_y),
                device_id_type=pl.DeviceIdType.MESH,
            )
            rdma.start()
            xf_rdmas.append(rdma)
        xk_rdmas = []
        for p in range(KP):
            rdma = pltpu.make_async_remote_copy(
                src_ref=xb.at[pl.ds(KBASE - sbase + p * KS, KS)],
                dst_ref=out_ref.at[pl.ds(my_x * m + KBASE + p * KS, KS)],
                send_sem=xk_send.at[p],
                recv_sem=xk_recv.at[p],
                device_id=(other_x, my_y),
                device_id_type=pl.DeviceIdType.MESH,
            )
            rdma.start()
            xk_rdmas.append(rdma)

        out_ref[pl.ds(my_x * m, m), :] = x_ref[...].astype(jnp.bfloat16)

        yf_rdmas = []
        for p in range(FP):
            xf_rdmas[p].wait()
            rdma = pltpu.make_async_remote_copy(
                src_ref=out_ref.at[pl.ds(obase + fb + p * FS, FS)],
                dst_ref=out_ref.at[pl.ds(obase + fb + p * FS, FS)],
                send_sem=yf_send.at[p],
                recv_sem=yf_recv.at[p],
                device_id=(my_x, other_y),
                device_id_type=pl.DeviceIdType.MESH,
            )
            rdma.start()
            yf_rdmas.append(rdma)

        for p in range(KP):
            xk_rdmas[p].wait()
        for p in range(FP):
            yf_rdmas[p].wait()

    return pl.pallas_call(
        body,
        out_shape=jax.ShapeDtypeStruct((2 * m, n), jnp.bfloat16),
        in_specs=[pl.BlockSpec(memory_space=pltpu.VMEM)],
        out_specs=pl.BlockSpec(memory_space=pltpu.VMEM),
        scratch_shapes=[
            pltpu.VMEM((SENT, n), jnp.bfloat16),
            pltpu.SemaphoreType.DMA((FP,)),
            pltpu.SemaphoreType.DMA((FP,)),
            pltpu.SemaphoreType.DMA((KP,)),
            pltpu.SemaphoreType.DMA((KP,)),
            pltpu.SemaphoreType.DMA((FP,)),
            pltpu.SemaphoreType.DMA((FP,)),
        ],
        compiler_params=pltpu.CompilerParams(collective_id=0),
    )(x)
